# baseline (speedup 1.0000x reference)
import functools

import jax
import jax.numpy as jnp
from jax import lax
from jax.experimental import pallas as pl
from jax.experimental.pallas import tpu as pltpu
from jax.experimental.pallas import tpu_sc as plsc

_B = 16
_R = 200
_EPG = 4000
_GPC = 8
_PADE = 4096
_NCHUNK = _PADE // 128
_RR = _R * _R
_K1 = 180
_K2 = 162
_KD = 8
_D1 = 32
_D2 = 32
_NEG = -1e30


def _adj_body(src_hbm, dst_hbm, w_hbm, zero_hbm, out_hbm,
              srcv, dstv, wv, idxv, zbuf, accsh, sem):
    c = lax.axis_index("c")
    s = lax.axis_index("s")
    g = c * _GPC + s

    @pl.when(s < _GPC)
    def _():
        eoff = g * _EPG
        pltpu.sync_copy(src_hbm.at[pl.ds(eoff, _EPG)], srcv.at[pl.ds(0, _EPG)])
        pltpu.sync_copy(dst_hbm.at[pl.ds(eoff, _EPG)], dstv.at[pl.ds(0, _EPG)])
        pltpu.sync_copy(w_hbm.at[pl.ds(eoff, _EPG)], wv.at[pl.ds(0, _EPG)])
        pltpu.sync_copy(zero_hbm, zbuf)
        pltpu.sync_copy(zbuf, accsh.at[pl.ds(s * _RR, _RR)])

        base = s * _RR
        goff = g * _R

        def body(i, carry):
            sl = pl.ds(i * 16, 16)
            sv = srcv[sl]
            dv = dstv[sl]
            uv = wv[sl]
            lane = i * 16 + lax.iota(jnp.int32, 16)
            m = lane < _EPG
            fi = base + (dv - goff) * _R + (sv - goff)
            idxv[i // 8, pl.ds((i % 8) * 16, 16)] = jnp.where(m, fi, base)
            wv[sl] = jnp.where(m, uv, 0.0)
            return carry

        lax.fori_loop(0, _PADE // 16, body, 0, unroll=2)

        for j in range(_NCHUNK):
            pltpu.async_copy(wv.at[pl.ds(j * 128, 128)],
                             accsh.at[idxv.at[j]], sem, add=True).wait()

        pltpu.sync_copy(accsh.at[pl.ds(s * _RR, _RR)], zbuf)
        pltpu.sync_copy(zbuf, out_hbm.at[pl.ds(g * _RR, _RR)])


def _build_adjacency(src, dst, w):
    zeros = jnp.zeros((_RR,), jnp.float32)
    mesh = plsc.VectorSubcoreMesh(core_axis_name="c", subcore_axis_name="s",
                                  num_cores=2, num_subcores=16)
    fn = pl.kernel(
        _adj_body,
        out_type=jax.ShapeDtypeStruct((_B * _RR,), jnp.float32),
        mesh=mesh,
        scratch_types=[
            pltpu.VMEM((_PADE,), jnp.int32),
            pltpu.VMEM((_PADE,), jnp.int32),
            pltpu.VMEM((_PADE,), jnp.float32),
            pltpu.VMEM((_NCHUNK, 128), jnp.int32),
            pltpu.VMEM((_RR,), jnp.float32),
            pltpu.VMEM_SHARED((_GPC * _RR,), jnp.float32),
            pltpu.SemaphoreType.DMA,
        ],
    )
    return fn(src, dst, w, zeros)


def _ddot(a, b, dims=((1,), (0,))):
    return lax.dot_general(a.astype(jnp.bfloat16), b.astype(jnp.bfloat16),
                           (dims, ((), ())),
                           preferred_element_type=jnp.float32)


def _bf16r(t):
    return t.astype(jnp.bfloat16).astype(jnp.float32)


def _sigmoid(t):
    return 1.0 / (1.0 + jnp.exp(-t))


def _rank(s_col, tie_col):
    s_mat = jnp.broadcast_to(s_col, (_R, _R))
    st_mat = jnp.broadcast_to(jnp.transpose(s_col), (_R, _R))
    t_mat = jnp.broadcast_to(tie_col, (_R, _R))
    tt_mat = jnp.broadcast_to(jnp.transpose(tie_col), (_R, _R))
    beat = jnp.where((st_mat > s_mat) |
                     ((st_mat == s_mat) & (tt_mat < t_mat)), 1.0, 0.0)
    return jnp.sum(beat, axis=1, keepdims=True)


def _pernode_contract(xb, wtab, in_c, in_pad, out_c):
    cols = []
    for o in range(out_c):
        cols.append(jnp.sum(xb * wtab[:, o * in_pad:o * in_pad + in_c],
                            axis=1, keepdims=True))
    return jnp.concatenate(cols, axis=1)


def _graph_body(xf_ref, a_ref, w1t1_ref, w2oi1_ref, b2oi1_ref,
                w1t2_ref, w2oi2_ref, b2oi2_ref,
                p1w_ref, p2w_ref, c1b_ref, c2b_ref, z_ref):
    xb = xf_ref[0]
    ab = a_ref[0]

    iota_r = lax.broadcasted_iota(jnp.int32, (_R, _R), 0)
    iota_c = lax.broadcasted_iota(jnp.int32, (_R, _R), 1)
    eye = jnp.where(iota_r == iota_c, 1.0, 0.0)
    p_col = lax.broadcasted_iota(jnp.int32, (_R, 1), 0).astype(jnp.float32)

    h1 = jnp.maximum(_bf16r(w1t1_ref[...]), 0.0)
    w1tab = _ddot(h1, w2oi1_ref[...]) + b2oi1_ref[...]
    xw1 = _pernode_contract(xb, w1tab, xb.shape[1], 256, _D1)
    out1 = _ddot(ab, xw1) + c1b_ref[...]

    p1w = p1w_ref[...]
    nrm1 = jnp.sqrt(jnp.sum(p1w * p1w, axis=0, keepdims=True)) + 1e-16
    s1 = _sigmoid(_ddot(out1, p1w) / nrm1)
    rank1 = _rank(s1, p_col)
    m1 = jnp.where(rank1 < float(_K1), 1.0, 0.0)
    m1r = jnp.transpose(m1)
    hp1 = out1 * s1 * m1
    x1max = jnp.max(jnp.where(m1 > 0.0, hp1, _NEG), axis=0, keepdims=True)
    x1mean = jnp.sum(hp1, axis=0, keepdims=True) / float(_K1)

    aa = ab * m1 * m1r + eye * m1
    a2 = _ddot(aa, aa) * (1.0 - eye)

    h2 = jnp.maximum(_bf16r(w1t2_ref[...]), 0.0)
    w2tab = _ddot(h2, w2oi2_ref[...]) + b2oi2_ref[...]
    xw2 = _pernode_contract(hp1, w2tab, _D1, 128, _D2)
    out2 = _ddot(a2, xw2) + c2b_ref[...]

    p2w = p2w_ref[...]
    nrm2 = jnp.sqrt(jnp.sum(p2w * p2w, axis=0, keepdims=True)) + 1e-16
    s2 = _sigmoid(_ddot(out2, p2w) / nrm2)
    s2 = jnp.where(m1 > 0.0, s2, -1.0)
    rank2 = _rank(s2, rank1)
    m2 = jnp.where(rank2 < float(_K2), 1.0, 0.0)
    hp2 = out2 * s2 * m2
    x2max = jnp.max(jnp.where(m2 > 0.0, hp2, _NEG), axis=0, keepdims=True)
    x2mean = jnp.sum(hp2, axis=0, keepdims=True) / float(_K2)

    z_ref[0] = jnp.concatenate([x1max, x1mean, x2max, x2mean], axis=1)


def _graph_stage(xf, a, w1t1, w2oi1, b2oi1, w1t2, w2oi2, b2oi2,
                 p1w, p2w, c1b, c2b):
    whole = lambda arr: pl.BlockSpec(arr.shape, lambda b: (0,) * arr.ndim)
    return pl.pallas_call(
        _graph_body,
        grid=(_B,),
        in_specs=[
            pl.BlockSpec((1, _R, xf.shape[2]), lambda b: (b, 0, 0)),
            pl.BlockSpec((1, _R, _R), lambda b: (b, 0, 0)),
            whole(w1t1), whole(w2oi1), whole(b2oi1),
            whole(w1t2), whole(w2oi2), whole(b2oi2),
            whole(p1w), whole(p2w), whole(c1b), whole(c2b),
        ],
        out_specs=pl.BlockSpec((1, 1, 4 * _D1), lambda b: (b, 0, 0)),
        out_shape=jax.ShapeDtypeStruct((_B, 1, 4 * _D1), jnp.float32),
    )(xf, a, w1t1, w2oi1, b2oi1, w1t2, w2oi2, b2oi2, p1w, p2w, c1b, c2b)


def _bn(y, g, b):
    mu = jnp.mean(y, axis=0, keepdims=True)
    var = jnp.mean((y - mu) * (y - mu), axis=0, keepdims=True)
    return (y - mu) / jnp.sqrt(var + 1e-5) * g + b


def _head_body(z_ref, w1_ref, b1_ref, g1_ref, e1_ref,
               w2_ref, b2_ref, g2_ref, e2_ref, w3_ref, b3_ref, out_ref):
    z = z_ref[...]
    y = jnp.maximum(_ddot(z, w1_ref[...], ((1,), (1,))) + b1_ref[...], 0.0)
    y = _bn(y, g1_ref[...], e1_ref[...])
    y = jnp.maximum(_ddot(y, w2_ref[...], ((1,), (1,))) + b2_ref[...], 0.0)
    y = _bn(y, g2_ref[...], e2_ref[...])
    out_ref[...] = _ddot(y, w3_ref[...], ((1,), (1,))) + b3_ref[...]


def _head_stage(z, w1, b1, g1, e1, w2, b2, g2, e2, w3, b3):
    return pl.pallas_call(
        _head_body,
        out_shape=jax.ShapeDtypeStruct((z.shape[0], w3.shape[0]), jnp.float32),
    )(z, w1, b1, g1, e1, w2, b2, g2, e2, w3, b3)


def kernel(x, edge_index, edge_weight, batch, pos,
           n1_w1, n1_w2, n1_b2, conv1_bias, pool1_w,
           n2_w1, n2_w2, n2_b2, conv2_bias, pool2_w,
           fc1_w, fc1_b, bn1_g, bn1_b, fc2_w, fc2_b, bn2_g, bn2_b,
           fc3_w, fc3_b):
    src = edge_index[0]
    dst = edge_index[1]

    a_flat = _build_adjacency(src, dst, edge_weight)
    a = a_flat.reshape(_B, _R, _R)

    indim = x.shape[1]
    xf = jnp.transpose(x, (0, 2, 1))

    pad1 = 256 - indim
    w2oi1 = jnp.pad(
        n1_w2.T.reshape(_KD, indim, _D1).transpose(0, 2, 1),
        ((0, 0), (0, 0), (0, pad1))).reshape(_KD, -1)
    b2oi1 = jnp.pad(n1_b2.reshape(indim, _D1).T,
                    ((0, 0), (0, pad1))).reshape(1, -1)
    pad2 = 128 - _D1
    w2oi2 = jnp.pad(
        n2_w2.T.reshape(_KD, _D1, _D2).transpose(0, 2, 1),
        ((0, 0), (0, 0), (0, pad2))).reshape(_KD, -1)
    b2oi2 = jnp.pad(n2_b2.reshape(_D1, _D2).T,
                    ((0, 0), (0, pad2))).reshape(1, -1)

    z = _graph_stage(xf, a, n1_w1.T, w2oi1, b2oi1, n2_w1.T, w2oi2, b2oi2,
                     pool1_w.reshape(_D1, 1), pool2_w.reshape(_D2, 1),
                     conv1_bias.reshape(1, _D1), conv2_bias.reshape(1, _D2))
    z = z.reshape(_B, 4 * _D1)

    return _head_stage(z, fc1_w, fc1_b.reshape(1, -1), bn1_g.reshape(1, -1),
                       bn1_b.reshape(1, -1), fc2_w, fc2_b.reshape(1, -1),
                       bn2_g.reshape(1, -1), bn2_b.reshape(1, -1),
                       fc3_w, fc3_b.reshape(1, -1))


# speedup vs baseline: 1.0161x; 1.0161x over previous
import functools

import jax
import jax.numpy as jnp
from jax import lax
from jax.experimental import pallas as pl
from jax.experimental.pallas import tpu as pltpu
from jax.experimental.pallas import tpu_sc as plsc

_B = 16
_R = 200
_EPG = 4000
_GPC = 8
_PADE = 4096
_NCHUNK = _PADE // 128
_RR = _R * _R
_K1 = 180
_K2 = 162
_KD = 8
_D1 = 32
_D2 = 32
_NEG = -1e30


def _adj_body(src_hbm, dst_hbm, w_hbm, zero_hbm, out_hbm,
              srcv, dstv, wv, idxv, zbuf, accsh, sem):
    c = lax.axis_index("c")
    s = lax.axis_index("s")
    g = c * _GPC + s

    @pl.when(s < _GPC)
    def _():
        eoff = g * _EPG
        pltpu.sync_copy(src_hbm.at[pl.ds(eoff, _EPG)], srcv.at[pl.ds(0, _EPG)])
        pltpu.sync_copy(dst_hbm.at[pl.ds(eoff, _EPG)], dstv.at[pl.ds(0, _EPG)])
        pltpu.sync_copy(w_hbm.at[pl.ds(eoff, _EPG)], wv.at[pl.ds(0, _EPG)])
        pltpu.sync_copy(zero_hbm, zbuf)
        pltpu.sync_copy(zbuf, accsh.at[pl.ds(s * _RR, _RR)])

        base = s * _RR
        goff = g * _R

        def body(i, carry):
            sl = pl.ds(i * 16, 16)
            sv = srcv[sl]
            dv = dstv[sl]
            uv = wv[sl]
            lane = i * 16 + lax.iota(jnp.int32, 16)
            m = lane < _EPG
            fi = base + (dv - goff) * _R + (sv - goff)
            idxv[i // 8, pl.ds((i % 8) * 16, 16)] = jnp.where(m, fi, base)
            wv[sl] = jnp.where(m, uv, 0.0)
            return carry

        lax.fori_loop(0, _PADE // 16, body, 0, unroll=2)

        for j in range(_NCHUNK):
            pltpu.async_copy(wv.at[pl.ds(j * 128, 128)],
                             accsh.at[idxv.at[j]], sem, add=True).wait()

        pltpu.sync_copy(accsh.at[pl.ds(s * _RR, _RR)], zbuf)
        pltpu.sync_copy(zbuf, out_hbm.at[pl.ds(g * _RR, _RR)])


def _build_adjacency(src, dst, w):
    zeros = jnp.zeros((_RR,), jnp.float32)
    mesh = plsc.VectorSubcoreMesh(core_axis_name="c", subcore_axis_name="s",
                                  num_cores=2, num_subcores=16)
    fn = pl.kernel(
        _adj_body,
        out_type=jax.ShapeDtypeStruct((_B * _RR,), jnp.float32),
        mesh=mesh,
        scratch_types=[
            pltpu.VMEM((_PADE,), jnp.int32),
            pltpu.VMEM((_PADE,), jnp.int32),
            pltpu.VMEM((_PADE,), jnp.float32),
            pltpu.VMEM((_NCHUNK, 128), jnp.int32),
            pltpu.VMEM((_RR,), jnp.float32),
            pltpu.VMEM_SHARED((_GPC * _RR,), jnp.float32),
            pltpu.SemaphoreType.DMA,
        ],
    )
    return fn(src, dst, w, zeros)


def _ddot(a, b, dims=((1,), (0,))):
    return lax.dot_general(a.astype(jnp.bfloat16), b.astype(jnp.bfloat16),
                           (dims, ((), ())),
                           preferred_element_type=jnp.float32)


def _bf16r(t):
    return t.astype(jnp.bfloat16).astype(jnp.float32)


def _sigmoid(t):
    return 1.0 / (1.0 + jnp.exp(-t))


def _rank(s_col, tie_col):
    s_mat = jnp.broadcast_to(s_col, (_R, _R))
    st_mat = jnp.broadcast_to(jnp.transpose(s_col), (_R, _R))
    t_mat = jnp.broadcast_to(tie_col, (_R, _R))
    tt_mat = jnp.broadcast_to(jnp.transpose(tie_col), (_R, _R))
    beat = jnp.where((st_mat > s_mat) |
                     ((st_mat == s_mat) & (tt_mat < t_mat)), 1.0, 0.0)
    return jnp.sum(beat, axis=1, keepdims=True)


def _pernode_contract(xb, wtab, in_c, in_pad, out_c):
    cols = []
    for o in range(out_c):
        cols.append(jnp.sum(xb * wtab[:, o * in_pad:o * in_pad + in_c],
                            axis=1, keepdims=True))
    return jnp.concatenate(cols, axis=1)


def _graph_body(x_ref, a_ref, w1t1_ref, w2oi1_ref, b2oi1_ref,
                w1t2_ref, w2oi2_ref, b2oi2_ref,
                p1w_ref, p2w_ref, c1b_ref, c2b_ref,
                fw1_ref, fb1_ref, g1_ref, e1_ref,
                fw2_ref, fb2_ref, g2_ref, e2_ref, fw3_ref, fb3_ref,
                out_ref, wt1_s, wt2_s, zacc):
    b = pl.program_id(0)
    xb = jnp.transpose(x_ref[0])
    ab = a_ref[0]

    iota_r = lax.broadcasted_iota(jnp.int32, (_R, _R), 0)
    iota_c = lax.broadcasted_iota(jnp.int32, (_R, _R), 1)
    eye = jnp.where(iota_r == iota_c, 1.0, 0.0)
    p_col = lax.broadcasted_iota(jnp.int32, (_R, 1), 0).astype(jnp.float32)

    @pl.when(b == 0)
    def _():
        h1 = jnp.maximum(_bf16r(w1t1_ref[...]), 0.0)
        wt1_s[...] = _ddot(h1, w2oi1_ref[...]) + b2oi1_ref[...]
        h2 = jnp.maximum(_bf16r(w1t2_ref[...]), 0.0)
        wt2_s[...] = _ddot(h2, w2oi2_ref[...]) + b2oi2_ref[...]

    xw1 = _pernode_contract(xb, wt1_s[...], _R, 256, _D1)
    out1 = _ddot(ab, xw1) + c1b_ref[...]

    p1w = p1w_ref[...]
    nrm1 = jnp.sqrt(jnp.sum(p1w * p1w, axis=0, keepdims=True)) + 1e-16
    s1 = _sigmoid(_ddot(out1, p1w) / nrm1)
    rank1 = _rank(s1, p_col)
    m1 = jnp.where(rank1 < float(_K1), 1.0, 0.0)
    m1r = jnp.transpose(m1)
    hp1 = out1 * s1 * m1
    x1max = jnp.max(jnp.where(m1 > 0.0, hp1, _NEG), axis=0, keepdims=True)
    x1mean = jnp.sum(hp1, axis=0, keepdims=True) / float(_K1)

    aa = ab * m1 * m1r + eye * m1
    a2 = _ddot(aa, aa) * (1.0 - eye)

    xw2 = _pernode_contract(hp1, wt2_s[...], _D1, 128, _D2)
    out2 = _ddot(a2, xw2) + c2b_ref[...]

    p2w = p2w_ref[...]
    nrm2 = jnp.sqrt(jnp.sum(p2w * p2w, axis=0, keepdims=True)) + 1e-16
    s2 = _sigmoid(_ddot(out2, p2w) / nrm2)
    s2 = jnp.where(m1 > 0.0, s2, -1.0)
    rank2 = _rank(s2, rank1)
    m2 = jnp.where(rank2 < float(_K2), 1.0, 0.0)
    hp2 = out2 * s2 * m2
    x2max = jnp.max(jnp.where(m2 > 0.0, hp2, _NEG), axis=0, keepdims=True)
    x2mean = jnp.sum(hp2, axis=0, keepdims=True) / float(_K2)

    zrow = jnp.concatenate([x1max, x1mean, x2max, x2mean], axis=1)
    row = lax.broadcasted_iota(jnp.int32, (_B, 4 * _D1), 0)
    zacc[...] = jnp.where(row == b, jnp.broadcast_to(zrow, (_B, 4 * _D1)),
                          zacc[...])

    @pl.when(b == _B - 1)
    def _():
        z = zacc[...]
        y = jnp.maximum(_ddot(z, fw1_ref[...], ((1,), (1,))) + fb1_ref[...], 0.0)
        y = _bn(y, g1_ref[...], e1_ref[...])
        y = jnp.maximum(_ddot(y, fw2_ref[...], ((1,), (1,))) + fb2_ref[...], 0.0)
        y = _bn(y, g2_ref[...], e2_ref[...])
        out_ref[...] = _ddot(y, fw3_ref[...], ((1,), (1,))) + fb3_ref[...]


def _graph_stage(x, a, w1t1, w2oi1, b2oi1, w1t2, w2oi2, b2oi2,
                 p1w, p2w, c1b, c2b, head_args):
    whole = lambda arr: pl.BlockSpec(arr.shape, lambda b: (0,) * arr.ndim)
    return pl.pallas_call(
        _graph_body,
        grid=(_B,),
        in_specs=[
            pl.BlockSpec((1, x.shape[1], _R), lambda b: (b, 0, 0)),
            pl.BlockSpec((1, _R, _R), lambda b: (b, 0, 0)),
            whole(w1t1), whole(w2oi1), whole(b2oi1),
            whole(w1t2), whole(w2oi2), whole(b2oi2),
            whole(p1w), whole(p2w), whole(c1b), whole(c2b),
        ] + [whole(h) for h in head_args],
        out_specs=pl.BlockSpec((_B, 2), lambda b: (0, 0)),
        out_shape=jax.ShapeDtypeStruct((_B, 2), jnp.float32),
        scratch_shapes=[
            pltpu.VMEM((_R, 32 * 256), jnp.float32),
            pltpu.VMEM((_R, 32 * 128), jnp.float32),
            pltpu.VMEM((_B, 4 * _D1), jnp.float32),
        ],
    )(x, a, w1t1, w2oi1, b2oi1, w1t2, w2oi2, b2oi2, p1w, p2w, c1b, c2b,
      *head_args)


def _bn(y, g, b):
    mu = jnp.mean(y, axis=0, keepdims=True)
    var = jnp.mean((y - mu) * (y - mu), axis=0, keepdims=True)
    return (y - mu) / jnp.sqrt(var + 1e-5) * g + b


def _head_body(z_ref, w1_ref, b1_ref, g1_ref, e1_ref,
               w2_ref, b2_ref, g2_ref, e2_ref, w3_ref, b3_ref, out_ref):
    z = z_ref[...]
    y = jnp.maximum(_ddot(z, w1_ref[...], ((1,), (1,))) + b1_ref[...], 0.0)
    y = _bn(y, g1_ref[...], e1_ref[...])
    y = jnp.maximum(_ddot(y, w2_ref[...], ((1,), (1,))) + b2_ref[...], 0.0)
    y = _bn(y, g2_ref[...], e2_ref[...])
    out_ref[...] = _ddot(y, w3_ref[...], ((1,), (1,))) + b3_ref[...]


def _head_stage(z, w1, b1, g1, e1, w2, b2, g2, e2, w3, b3):
    return pl.pallas_call(
        _head_body,
        out_shape=jax.ShapeDtypeStruct((z.shape[0], w3.shape[0]), jnp.float32),
    )(z, w1, b1, g1, e1, w2, b2, g2, e2, w3, b3)


def kernel(x, edge_index, edge_weight, batch, pos,
           n1_w1, n1_w2, n1_b2, conv1_bias, pool1_w,
           n2_w1, n2_w2, n2_b2, conv2_bias, pool2_w,
           fc1_w, fc1_b, bn1_g, bn1_b, fc2_w, fc2_b, bn2_g, bn2_b,
           fc3_w, fc3_b):
    src = edge_index[0]
    dst = edge_index[1]

    a_flat = _build_adjacency(src, dst, edge_weight)
    a = a_flat.reshape(_B, _R, _R)

    indim = x.shape[1]

    pad1 = 256 - indim
    w2oi1 = jnp.pad(
        n1_w2.T.reshape(_KD, indim, _D1).transpose(0, 2, 1),
        ((0, 0), (0, 0), (0, pad1))).reshape(_KD, -1)
    b2oi1 = jnp.pad(n1_b2.reshape(indim, _D1).T,
                    ((0, 0), (0, pad1))).reshape(1, -1)
    pad2 = 128 - _D1
    w2oi2 = jnp.pad(
        n2_w2.T.reshape(_KD, _D1, _D2).transpose(0, 2, 1),
        ((0, 0), (0, 0), (0, pad2))).reshape(_KD, -1)
    b2oi2 = jnp.pad(n2_b2.reshape(_D1, _D2).T,
                    ((0, 0), (0, pad2))).reshape(1, -1)

    head_args = [fc1_w, fc1_b.reshape(1, -1), bn1_g.reshape(1, -1),
                 bn1_b.reshape(1, -1), fc2_w, fc2_b.reshape(1, -1),
                 bn2_g.reshape(1, -1), bn2_b.reshape(1, -1),
                 fc3_w, fc3_b.reshape(1, -1)]
    return _graph_stage(x, a, n1_w1.T, w2oi1, b2oi1, n2_w1.T, w2oi2, b2oi2,
                        pool1_w.reshape(_D1, 1), pool2_w.reshape(_D2, 1),
                        conv1_bias.reshape(1, _D1), conv2_bias.reshape(1, _D2),
                        head_args)
